# baseline (speedup 1.0000x reference)
import functools

import jax
import jax.numpy as jnp
from jax import lax
from jax.experimental import pallas as pl
from jax.experimental.pallas import tpu as pltpu
from jax.experimental.pallas import tpu_sc as plsc

BS, C, H, W = 16, 512, 64, 64
HW = H * W
CH = C // 2
KP = 64
GR = 8
NG = CH // GR


def _sc_masked_mean_body(value_hbm, lay_hbm, out_hbm,
                         vbuf_a, vbuf_b, mbuf, obuf, sem_a, sem_b):
    core = lax.axis_index("c")
    sub = lax.axis_index("s")
    b = sub
    lo = core * CH

    pltpu.sync_copy(lay_hbm.at[b], mbuf)

    def cnt_step(i, acc):
        return acc + mbuf[pl.ds(i * 16, 16)]

    cntv = lax.fori_loop(0, HW // 16, cnt_step, jnp.zeros(16, jnp.float32))
    cnt = cntv[0]
    for j in range(1, 16):
        cnt = cnt + cntv[j]
    cnt_b = jnp.zeros(16, jnp.float32) + cnt
    scale = jnp.where(cnt_b > 0.0,
                      jnp.ones(16, jnp.float32) / jnp.maximum(cnt_b, 1.0),
                      jnp.zeros(16, jnp.float32))

    lane = lax.iota(jnp.int32, 16)

    def src_slice(g):
        return value_hbm.at[b, pl.ds(lo + g * GR, GR), :]

    def row_sums(vbuf):
        def inner(i, accs):
            m = mbuf[pl.ds(i * 16, 16)]
            return tuple(accs[j] + vbuf[j, pl.ds(i * 16, 16)] * m
                         for j in range(GR))

        accs = lax.fori_loop(0, HW // 16, inner,
                             tuple(jnp.zeros(16, jnp.float32)
                                   for _ in range(GR)))
        sums = []
        for j in range(GR):
            a = accs[j]
            s = a[0]
            for i in range(1, 16):
                s = s + a[i]
            sums.append(s)
        return sums

    pltpu.async_copy(src_slice(0), vbuf_a, sem_a)
    pltpu.async_copy(src_slice(1), vbuf_b, sem_b)

    def pair(k, carry):
        ga = k * 2
        pltpu.make_async_copy(src_slice(ga), vbuf_a, sem_a).wait()
        sums_a = row_sums(vbuf_a)

        @pl.when(k < NG // 2 - 1)
        def _():
            pltpu.async_copy(src_slice(ga + 2), vbuf_a, sem_a)

        pltpu.make_async_copy(src_slice(ga + 1), vbuf_b, sem_b).wait()
        sums_b = row_sums(vbuf_b)

        @pl.when(k < NG // 2 - 1)
        def _():
            pltpu.async_copy(src_slice(ga + 3), vbuf_b, sem_b)

        tsum = jnp.zeros(16, jnp.float32)
        for j, s in enumerate(sums_a + sums_b):
            tsum = jnp.where(lane == j, jnp.zeros(16, jnp.float32) + s, tsum)
        obuf[pl.ds(k * 16, 16)] = tsum * scale
        return carry

    lax.fori_loop(0, NG // 2, pair, 0)
    pltpu.sync_copy(obuf, out_hbm.at[b, pl.ds(lo, CH)])


def _sc_masked_mean(value_r, lay):
    mesh = plsc.VectorSubcoreMesh(core_axis_name="c", subcore_axis_name="s")
    run = functools.partial(
        pl.kernel,
        mesh=mesh,
        out_type=jax.ShapeDtypeStruct((BS, C), jnp.float32),
        scratch_types=[
            pltpu.VMEM((GR, HW), jnp.float32),
            pltpu.VMEM((GR, HW), jnp.float32),
            pltpu.VMEM((HW,), jnp.float32),
            pltpu.VMEM((CH,), jnp.float32),
            pltpu.SemaphoreType.DMA,
            pltpu.SemaphoreType.DMA,
        ],
    )(_sc_masked_mean_body)
    return run(value_r.reshape(BS, C, HW), lay.reshape(BS, HW))


def _fused_body(query_ref, wq_ref, saw_ref, w2_ref, bias_ref, wl_ref, bl_ref,
                key_ref, layouts_ref, km_ref, cum_ref,
                logit_ref, sawo_ref, cumo_ref, p_scr):
    ct = pl.program_id(1)

    @pl.when(ct == 0)
    def _():
        pos = jax.lax.broadcasted_iota(jnp.int32, (1, HW), 1)
        hh = pos // W
        ww = pos % W
        mh = {d: ((hh + d >= 0) & (hh + d < H)).astype(jnp.float32)
              for d in range(-2, 3)}
        mw = {d: ((ww + d >= 0) & (ww + d < W)).astype(jnp.float32)
              for d in range(-2, 3)}
        k = 0
        for src_ref in (saw_ref, cum_ref):
            s0 = src_ref[0]
            for dy in range(-2, 3):
                for dx in range(-2, 3):
                    s = dy * W + dx
                    rolled = s0 if s == 0 else jnp.roll(s0, -s, axis=1)
                    p_scr[k:k + 1, :] = rolled * mh[dy] * mw[dx]
                    k += 1
        p_scr[50:KP, :] = jnp.zeros((KP - 50, HW), jnp.float32)
        cumo_ref[0] = jnp.minimum(layouts_ref[0] + cum_ref[0], 1.0)

    dlo = ct * CH
    wq_t = wq_ref[pl.ds(dlo, CH), :]
    qv = jax.lax.dot_general(wq_t, query_ref[0],
                             (((1,), (1,)), ((), ())),
                             preferred_element_type=jnp.float32)
    w2_t = w2_ref[pl.ds(dlo, CH), :]
    conv = jax.lax.dot_general(w2_t, p_scr[...],
                               (((1,), (0,)), ((), ())),
                               preferred_element_type=jnp.float32)
    fusion = key_ref[0] + conv + qv + bias_ref[pl.ds(dlo, CH), :]
    t = jnp.tanh(fusion)
    part = jax.lax.dot_general(wl_ref[:, pl.ds(dlo, CH)], t,
                               (((1,), (0,)), ((), ())),
                               preferred_element_type=jnp.float32)

    @pl.when(ct == 0)
    def _():
        logit_ref[0] = part + bl_ref[0, 0]

    @pl.when(ct == 1)
    def _():
        logit = logit_ref[0] + part
        logit_ref[0] = logit
        l = logit - (1.0 - km_ref[0]) * 100000000.0
        m = jnp.max(l, axis=1, keepdims=True)
        e = jnp.exp(l - m)
        sawo_ref[0] = e / jnp.sum(e, axis=1, keepdims=True)


def kernel(key, key_mask, query, spatial_att_weight, cum_spatial_att_weight,
           value, state, layouts, Wq, bq, Ww, bw, Wc, bc, Wl, bl):
    key_r = key.reshape(BS, C, HW)
    km = key_mask.reshape(BS, 1, HW)
    saw = spatial_att_weight.reshape(BS, 1, HW)
    cum = cum_spatial_att_weight.reshape(BS, 1, HW)
    lay = layouts.reshape(BS, 1, HW)
    query_r = query.reshape(BS, 1, C)

    W2 = jnp.concatenate([Ww.reshape(C, 25), Wc.reshape(C, 25),
                          jnp.zeros((C, KP - 50), jnp.float32)], axis=1)
    bias = (bq + bw + bc).reshape(C, 1)
    Wl2 = Wl.reshape(1, C)

    grid = (BS, 2)
    out_shape = [
        jax.ShapeDtypeStruct((BS, 1, HW), jnp.float32),
        jax.ShapeDtypeStruct((BS, 1, HW), jnp.float32),
        jax.ShapeDtypeStruct((BS, 1, HW), jnp.float32),
    ]
    in_specs = [
        pl.BlockSpec((1, 1, C), lambda b, c: (b, 0, 0)),
        pl.BlockSpec((C, C), lambda b, c: (0, 0)),
        pl.BlockSpec((1, 1, HW), lambda b, c: (b, 0, 0)),
        pl.BlockSpec((C, KP), lambda b, c: (0, 0)),
        pl.BlockSpec((C, 1), lambda b, c: (0, 0)),
        pl.BlockSpec((1, C), lambda b, c: (0, 0)),
        pl.BlockSpec((1, 1), lambda b, c: (0, 0)),
        pl.BlockSpec((1, CH, HW), lambda b, c: (b, c, 0)),
        pl.BlockSpec((1, 1, HW), lambda b, c: (b, 0, 0)),
        pl.BlockSpec((1, 1, HW), lambda b, c: (b, 0, 0)),
        pl.BlockSpec((1, 1, HW), lambda b, c: (b, 0, 0)),
    ]
    out_specs = [
        pl.BlockSpec((1, 1, HW), lambda b, c: (b, 0, 0)),
        pl.BlockSpec((1, 1, HW), lambda b, c: (b, 0, 0)),
        pl.BlockSpec((1, 1, HW), lambda b, c: (b, 0, 0)),
    ]
    logit, sawo, cumo = pl.pallas_call(
        _fused_body,
        grid=grid,
        in_specs=in_specs,
        out_specs=out_specs,
        out_shape=out_shape,
        scratch_shapes=[pltpu.VMEM((KP, HW), jnp.float32)],
        compiler_params=pltpu.CompilerParams(
            dimension_semantics=("arbitrary", "arbitrary")),
    )(query_r, Wq, saw, W2, bias, Wl2, bl.reshape(1, 1),
      key_r, lay, km, cum)

    outputs = _sc_masked_mean(value, layouts)

    return (state,
            outputs,
            logit.reshape(BS, 1, H, W),
            sawo.reshape(BS, 1, H, W),
            cumo.reshape(BS, 1, H, W))


# speedup vs baseline: 1.0537x; 1.0537x over previous
import functools

import jax
import jax.numpy as jnp
from jax import lax
from jax.experimental import pallas as pl
from jax.experimental.pallas import tpu as pltpu
from jax.experimental.pallas import tpu_sc as plsc

BS, C, H, W = 16, 512, 64, 64
HW = H * W
CH = C // 2
KP = 64
GR = 8
NG = CH // GR


def _sc_masked_mean_body(value_hbm, lay_hbm, out_hbm,
                         vbuf_a, vbuf_b, mbuf, obuf, sem_a, sem_b):
    core = lax.axis_index("c")
    sub = lax.axis_index("s")
    b = sub
    lo = core * CH

    pltpu.sync_copy(lay_hbm.at[b], mbuf)

    def cnt_step(i, acc):
        return acc + mbuf[pl.ds(i * 16, 16)]

    cntv = lax.fori_loop(0, HW // 16, cnt_step, jnp.zeros(16, jnp.float32))
    cnt = cntv[0]
    for j in range(1, 16):
        cnt = cnt + cntv[j]
    cnt_b = jnp.zeros(16, jnp.float32) + cnt
    scale = jnp.where(cnt_b > 0.0,
                      jnp.ones(16, jnp.float32) / jnp.maximum(cnt_b, 1.0),
                      jnp.zeros(16, jnp.float32))

    lane = lax.iota(jnp.int32, 16)

    def src_slice(g):
        return value_hbm.at[b, pl.ds(lo + g * GR, GR), :]

    def row_sums(vbuf):
        def inner(i, accs):
            m = mbuf[pl.ds(i * 16, 16)]
            return tuple(accs[j] + vbuf[j, pl.ds(i * 16, 16)] * m
                         for j in range(GR))

        accs = lax.fori_loop(0, HW // 16, inner,
                             tuple(jnp.zeros(16, jnp.float32)
                                   for _ in range(GR)))
        sums = []
        for j in range(GR):
            a = accs[j]
            s = a[0]
            for i in range(1, 16):
                s = s + a[i]
            sums.append(s)
        return sums

    pltpu.async_copy(src_slice(0), vbuf_a, sem_a)
    pltpu.async_copy(src_slice(1), vbuf_b, sem_b)

    def pair(k, carry):
        ga = k * 2
        pltpu.make_async_copy(src_slice(ga), vbuf_a, sem_a).wait()
        sums_a = row_sums(vbuf_a)

        @pl.when(k < NG // 2 - 1)
        def _():
            pltpu.async_copy(src_slice(ga + 2), vbuf_a, sem_a)

        pltpu.make_async_copy(src_slice(ga + 1), vbuf_b, sem_b).wait()
        sums_b = row_sums(vbuf_b)

        @pl.when(k < NG // 2 - 1)
        def _():
            pltpu.async_copy(src_slice(ga + 3), vbuf_b, sem_b)

        tsum = jnp.zeros(16, jnp.float32)
        for j, s in enumerate(sums_a + sums_b):
            tsum = jnp.where(lane == j, jnp.zeros(16, jnp.float32) + s, tsum)
        obuf[pl.ds(k * 16, 16)] = tsum * scale
        return carry

    lax.fori_loop(0, NG // 2, pair, 0)
    pltpu.sync_copy(obuf, out_hbm.at[b, pl.ds(lo, CH)])


def _sc_masked_mean(value_r, lay):
    mesh = plsc.VectorSubcoreMesh(core_axis_name="c", subcore_axis_name="s")
    run = functools.partial(
        pl.kernel,
        mesh=mesh,
        out_type=jax.ShapeDtypeStruct((BS, C), jnp.float32),
        scratch_types=[
            pltpu.VMEM((GR, HW), jnp.float32),
            pltpu.VMEM((GR, HW), jnp.float32),
            pltpu.VMEM((HW,), jnp.float32),
            pltpu.VMEM((CH,), jnp.float32),
            pltpu.SemaphoreType.DMA,
            pltpu.SemaphoreType.DMA,
        ],
    )(_sc_masked_mean_body)
    return run(value_r.reshape(BS, C, HW), lay.reshape(BS, HW))


def _fused_body(query_ref, wq_ref, saw_ref, w2_ref, bias_ref, wl_ref, bl_ref,
                key0_ref, key1_ref, layouts_ref, km_ref, cum_ref,
                logit_ref, sawo_ref, cumo_ref, p_scr):
    pos = jax.lax.broadcasted_iota(jnp.int32, (1, HW), 1)
    hh = pos // W
    ww = pos % W
    mh = {d: ((hh + d >= 0) & (hh + d < H)).astype(jnp.float32)
          for d in range(-2, 3)}
    mw = {d: ((ww + d >= 0) & (ww + d < W)).astype(jnp.float32)
          for d in range(-2, 3)}
    k = 0
    for src_ref in (saw_ref, cum_ref):
        src = src_ref[0]
        for dy in range(-2, 3):
            for dx in range(-2, 3):
                s = dy * W + dx
                rolled = src if s == 0 else jnp.roll(src, -s, axis=1)
                p_scr[k:k + 1, :] = rolled * mh[dy] * mw[dx]
                k += 1
    p_scr[50:KP, :] = jnp.zeros((KP - 50, HW), jnp.float32)

    def half(key_ref, lo):
        sl = slice(lo, lo + CH)
        qv = jax.lax.dot_general(wq_ref[sl, :], query_ref[0],
                                 (((1,), (1,)), ((), ())),
                                 preferred_element_type=jnp.float32)
        conv = jax.lax.dot_general(w2_ref[sl, :], p_scr[...],
                                   (((1,), (0,)), ((), ())),
                                   preferred_element_type=jnp.float32)
        fusion = key_ref[0] + conv + qv + bias_ref[sl, :]
        t = jnp.tanh(fusion)
        return jax.lax.dot_general(wl_ref[:, sl], t, (((1,), (0,)), ((), ())),
                                   preferred_element_type=jnp.float32)

    logit = half(key0_ref, 0) + half(key1_ref, CH) + bl_ref[0, 0]
    logit_ref[0] = logit

    l = logit - (1.0 - km_ref[0]) * 100000000.0
    m = jnp.max(l, axis=1, keepdims=True)
    e = jnp.exp(l - m)
    sawo_ref[0] = e / jnp.sum(e, axis=1, keepdims=True)

    cumo_ref[0] = jnp.minimum(layouts_ref[0] + cum_ref[0], 1.0)


def kernel(key, key_mask, query, spatial_att_weight, cum_spatial_att_weight,
           value, state, layouts, Wq, bq, Ww, bw, Wc, bc, Wl, bl):
    key_r = key.reshape(BS, C, HW)
    km = key_mask.reshape(BS, 1, HW)
    saw = spatial_att_weight.reshape(BS, 1, HW)
    cum = cum_spatial_att_weight.reshape(BS, 1, HW)
    lay = layouts.reshape(BS, 1, HW)
    query_r = query.reshape(BS, 1, C)

    W2 = jnp.concatenate([Ww.reshape(C, 25), Wc.reshape(C, 25),
                          jnp.zeros((C, KP - 50), jnp.float32)], axis=1)
    bias = (bq + bw + bc).reshape(C, 1)
    Wl2 = Wl.reshape(1, C)

    grid = (BS,)
    out_shape = [
        jax.ShapeDtypeStruct((BS, 1, HW), jnp.float32),
        jax.ShapeDtypeStruct((BS, 1, HW), jnp.float32),
        jax.ShapeDtypeStruct((BS, 1, HW), jnp.float32),
    ]
    in_specs = [
        pl.BlockSpec((1, 1, C), lambda b: (b, 0, 0)),
        pl.BlockSpec((C, C), lambda b: (0, 0)),
        pl.BlockSpec((1, 1, HW), lambda b: (b, 0, 0)),
        pl.BlockSpec((C, KP), lambda b: (0, 0)),
        pl.BlockSpec((C, 1), lambda b: (0, 0)),
        pl.BlockSpec((1, C), lambda b: (0, 0)),
        pl.BlockSpec((1, 1), lambda b: (0, 0)),
        pl.BlockSpec((1, CH, HW), lambda b: (b, 0, 0)),
        pl.BlockSpec((1, CH, HW), lambda b: (b, 1, 0)),
        pl.BlockSpec((1, 1, HW), lambda b: (b, 0, 0)),
        pl.BlockSpec((1, 1, HW), lambda b: (b, 0, 0)),
        pl.BlockSpec((1, 1, HW), lambda b: (b, 0, 0)),
    ]
    out_specs = [
        pl.BlockSpec((1, 1, HW), lambda b: (b, 0, 0)),
        pl.BlockSpec((1, 1, HW), lambda b: (b, 0, 0)),
        pl.BlockSpec((1, 1, HW), lambda b: (b, 0, 0)),
    ]
    logit, sawo, cumo = pl.pallas_call(
        _fused_body,
        grid=grid,
        in_specs=in_specs,
        out_specs=out_specs,
        out_shape=out_shape,
        scratch_shapes=[pltpu.VMEM((KP, HW), jnp.float32)],
        compiler_params=pltpu.CompilerParams(
            dimension_semantics=("arbitrary",)),
    )(query_r, Wq, saw, W2, bias, Wl2, bl.reshape(1, 1),
      key_r, key_r, lay, km, cum)

    outputs = _sc_masked_mean(value, layouts)

    return (state,
            outputs,
            logit.reshape(BS, 1, H, W),
            sawo.reshape(BS, 1, H, W),
            cumo.reshape(BS, 1, H, W))


# speedup vs baseline: 1.0545x; 1.0007x over previous
import functools

import jax
import jax.numpy as jnp
from jax import lax
from jax.experimental import pallas as pl
from jax.experimental.pallas import tpu as pltpu
from jax.experimental.pallas import tpu_sc as plsc

BS, C, H, W = 16, 512, 64, 64
HW = H * W
CH = C // 2
KP = 64
GR = 8
NG = CH // GR


def _sc_masked_mean_body(value_hbm, lay_hbm, out_hbm,
                         vbuf_a, vbuf_b, mbuf, obuf, sem_a, sem_b):
    core = lax.axis_index("c")
    sub = lax.axis_index("s")
    b = sub
    lo = core * CH

    pltpu.sync_copy(lay_hbm.at[b], mbuf)

    def cnt_step(i, acc):
        return acc + mbuf[pl.ds(i * 16, 16)]

    cntv = lax.fori_loop(0, HW // 16, cnt_step, jnp.zeros(16, jnp.float32))
    cnt = cntv[0]
    for j in range(1, 16):
        cnt = cnt + cntv[j]
    cnt_b = jnp.zeros(16, jnp.float32) + cnt
    scale = jnp.where(cnt_b > 0.0,
                      jnp.ones(16, jnp.float32) / jnp.maximum(cnt_b, 1.0),
                      jnp.zeros(16, jnp.float32))

    lane = lax.iota(jnp.int32, 16)

    def src_slice(g):
        return value_hbm.at[b, pl.ds(lo + g * GR, GR), :]

    def row_sums(vbuf):
        def inner(i, accs):
            m = mbuf[pl.ds(i * 16, 16)]
            return tuple(accs[j] + vbuf[j, pl.ds(i * 16, 16)] * m
                         for j in range(GR))

        accs = lax.fori_loop(0, HW // 16, inner,
                             tuple(jnp.zeros(16, jnp.float32)
                                   for _ in range(GR)))
        sums = []
        for j in range(GR):
            a = accs[j]
            s = a[0]
            for i in range(1, 16):
                s = s + a[i]
            sums.append(s)
        return sums

    pltpu.async_copy(src_slice(0), vbuf_a, sem_a)
    pltpu.async_copy(src_slice(1), vbuf_b, sem_b)

    def pair(k, carry):
        ga = k * 2
        pltpu.make_async_copy(src_slice(ga), vbuf_a, sem_a).wait()
        sums_a = row_sums(vbuf_a)

        @pl.when(k < NG // 2 - 1)
        def _():
            pltpu.async_copy(src_slice(ga + 2), vbuf_a, sem_a)

        pltpu.make_async_copy(src_slice(ga + 1), vbuf_b, sem_b).wait()
        sums_b = row_sums(vbuf_b)

        @pl.when(k < NG // 2 - 1)
        def _():
            pltpu.async_copy(src_slice(ga + 3), vbuf_b, sem_b)

        tsum = jnp.zeros(16, jnp.float32)
        for j, s in enumerate(sums_a + sums_b):
            tsum = jnp.where(lane == j, jnp.zeros(16, jnp.float32) + s, tsum)
        obuf[pl.ds(k * 16, 16)] = tsum * scale
        return carry

    lax.fori_loop(0, NG // 2, pair, 0)
    pltpu.sync_copy(obuf, out_hbm.at[b, pl.ds(lo, CH)])


def _sc_masked_mean(value_r, lay):
    mesh = plsc.VectorSubcoreMesh(core_axis_name="c", subcore_axis_name="s")
    run = functools.partial(
        pl.kernel,
        mesh=mesh,
        out_type=jax.ShapeDtypeStruct((BS, C), jnp.float32),
        scratch_types=[
            pltpu.VMEM((GR, HW), jnp.float32),
            pltpu.VMEM((GR, HW), jnp.float32),
            pltpu.VMEM((HW,), jnp.float32),
            pltpu.VMEM((CH,), jnp.float32),
            pltpu.SemaphoreType.DMA,
            pltpu.SemaphoreType.DMA,
        ],
    )(_sc_masked_mean_body)
    return run(value_r.reshape(BS, C, HW), lay.reshape(BS, HW))


def _fused_body(query_ref, wq_ref, saw_ref, w2_ref, bias_ref, wl_ref, bl_ref,
                key0_ref, key1_ref, layouts_ref, km_ref, cum_ref,
                logit_ref, sawo_ref, cumo_ref, p_scr):
    pos = jax.lax.broadcasted_iota(jnp.int32, (1, HW), 1)
    hh = pos // W
    ww = pos % W
    mh = {d: ((hh + d >= 0) & (hh + d < H)).astype(jnp.float32)
          for d in range(-2, 3)}
    mw = {d: ((ww + d >= 0) & (ww + d < W)).astype(jnp.float32)
          for d in range(-2, 3)}
    k = 0
    for src_ref in (saw_ref, cum_ref):
        src = src_ref[0]
        for dy in range(-2, 3):
            for dx in range(-2, 3):
                s = dy * W + dx
                rolled = src if s == 0 else jnp.roll(src, -s, axis=1)
                p_scr[k:k + 1, :] = rolled * mh[dy] * mw[dx]
                k += 1
    p_scr[50:KP, :] = jnp.zeros((KP - 50, HW), jnp.float32)

    def half(key_ref, lo):
        sl = slice(lo, lo + CH)
        qv = jax.lax.dot_general(wq_ref[sl, :], query_ref[0],
                                 (((1,), (1,)), ((), ())),
                                 preferred_element_type=jnp.float32)
        conv = jax.lax.dot_general(w2_ref[sl, :], p_scr[...],
                                   (((1,), (0,)), ((), ())),
                                   preferred_element_type=jnp.float32)
        fusion = key_ref[0] + conv + qv + bias_ref[sl, :]
        t = jnp.tanh(fusion)
        return jax.lax.dot_general(wl_ref[:, sl], t, (((1,), (0,)), ((), ())),
                                   preferred_element_type=jnp.float32)

    logit = half(key0_ref, 0) + half(key1_ref, CH) + bl_ref[0, 0]
    logit_ref[0] = logit

    l = logit - (1.0 - km_ref[0]) * 100000000.0
    m = jnp.max(l, axis=1, keepdims=True)
    e = jnp.exp(l - m)
    sawo_ref[0] = e / jnp.sum(e, axis=1, keepdims=True)

    cumo_ref[0] = jnp.minimum(layouts_ref[0] + cum_ref[0], 1.0)


def kernel(key, key_mask, query, spatial_att_weight, cum_spatial_att_weight,
           value, state, layouts, Wq, bq, Ww, bw, Wc, bc, Wl, bl):
    key_r = key.reshape(BS, C, HW)
    km = key_mask.reshape(BS, 1, HW)
    saw = spatial_att_weight.reshape(BS, 1, HW)
    cum = cum_spatial_att_weight.reshape(BS, 1, HW)
    lay = layouts.reshape(BS, 1, HW)
    query_r = query.reshape(BS, 1, C)

    W2 = jnp.concatenate([Ww.reshape(C, 25), Wc.reshape(C, 25),
                          jnp.zeros((C, KP - 50), jnp.float32)], axis=1)
    bias = (bq + bw + bc).reshape(C, 1)
    Wl2 = Wl.reshape(1, C)

    grid = (BS,)
    out_shape = [
        jax.ShapeDtypeStruct((BS, 1, HW), jnp.float32),
        jax.ShapeDtypeStruct((BS, 1, HW), jnp.float32),
        jax.ShapeDtypeStruct((BS, 1, HW), jnp.float32),
    ]
    in_specs = [
        pl.BlockSpec((1, 1, C), lambda b: (b, 0, 0)),
        pl.BlockSpec((C, C), lambda b: (0, 0)),
        pl.BlockSpec((1, 1, HW), lambda b: (b, 0, 0)),
        pl.BlockSpec((C, KP), lambda b: (0, 0)),
        pl.BlockSpec((C, 1), lambda b: (0, 0)),
        pl.BlockSpec((1, C), lambda b: (0, 0)),
        pl.BlockSpec((1, 1), lambda b: (0, 0)),
        pl.BlockSpec((1, CH, HW), lambda b: (b, 0, 0)),
        pl.BlockSpec((1, CH, HW), lambda b: (b, 1, 0)),
        pl.BlockSpec((1, 1, HW), lambda b: (b, 0, 0)),
        pl.BlockSpec((1, 1, HW), lambda b: (b, 0, 0)),
        pl.BlockSpec((1, 1, HW), lambda b: (b, 0, 0)),
    ]
    out_specs = [
        pl.BlockSpec((1, 1, HW), lambda b: (b, 0, 0)),
        pl.BlockSpec((1, 1, HW), lambda b: (b, 0, 0)),
        pl.BlockSpec((1, 1, HW), lambda b: (b, 0, 0)),
    ]
    logit, sawo, cumo = pl.pallas_call(
        _fused_body,
        grid=grid,
        in_specs=in_specs,
        out_specs=out_specs,
        out_shape=out_shape,
        scratch_shapes=[pltpu.VMEM((KP, HW), jnp.float32)],
        compiler_params=pltpu.CompilerParams(
            dimension_semantics=("parallel",)),
    )(query_r, Wq, saw, W2, bias, Wl2, bl.reshape(1, 1),
      key_r, key_r, lay, km, cum)

    outputs = _sc_masked_mean(value, layouts)

    return (state,
            outputs,
            logit.reshape(BS, 1, H, W),
            sawo.reshape(BS, 1, H, W),
            cumo.reshape(BS, 1, H, W))
